# baseline (speedup 1.0000x reference)
import functools

import jax
import jax.numpy as jnp
from jax import lax
from jax.experimental import pallas as pl
from jax.experimental.pallas import tpu as pltpu
from jax.experimental.pallas import tpu_sc as plsc

_NUM_CORES = 2
_NUM_SUBCORES = 16
_NUM_WORKERS = _NUM_CORES * _NUM_SUBCORES
_CHUNK = 512
_LANE = 128
_SUB = 8


@functools.lru_cache(maxsize=None)
def _make_gather(n_b, n_t, d):
    n = n_b * n_t
    per_w = n // _NUM_WORKERS
    n_chunks = per_w // _CHUNK
    assert n_chunks % 2 == 0
    assert n_b % _LANE == 0 and d % _SUB == 0 and _CHUNK % _LANE == 0
    n_fb = d // _SUB
    tile_words = _SUB * _LANE
    units = _CHUNK // _LANE
    mesh = plsc.VectorSubcoreMesh(
        core_axis_name="c",
        subcore_axis_name="s",
        num_cores=_NUM_CORES,
        num_subcores=_NUM_SUBCORES,
    )

    @functools.partial(
        pl.kernel,
        mesh=mesh,
        compiler_params=pltpu.CompilerParams(
            use_tc_tiling_on_sc=False, needs_layout_passes=False),
        out_type=jax.ShapeDtypeStruct((n * d,), jnp.float32),
        scratch_types=[
            pltpu.VMEM((per_w,), jnp.int32),
            pltpu.VMEM((2, _CHUNK, d), jnp.float32),
            pltpu.VMEM((2, _CHUNK * d), jnp.float32),
            pltpu.SemaphoreType.DMA,
            pltpu.SemaphoreType.DMA,
            pltpu.SemaphoreType.DMA,
            pltpu.SemaphoreType.DMA,
        ],
    )
    def gather_kernel(idx_hbm, table_hbm, out_hbm, idx_v, rows_v, tile_v,
                      g0, g1, w0, w1):
        gsem = (g0, g1)
        wsem = (w0, w1)
        wid = lax.axis_index("s") * _NUM_CORES + lax.axis_index("c")
        base = wid * per_w

        pltpu.sync_copy(idx_hbm.at[pl.ds(pl.multiple_of(base, 8), per_w)], idx_v)

        io = lax.iota(jnp.int32, 16)
        blk = units * tile_words

        def issue_gather(c, s):
            pltpu.async_copy(
                table_hbm.at[idx_v.at[pl.ds(c * _CHUNK, _CHUNK)]],
                rows_v.at[s], gsem[s])

        def wait_gather(s):
            pltpu.make_async_copy(
                table_hbm.at[idx_v.at[pl.ds(0, _CHUNK)]], rows_v.at[s], gsem[s]
            ).wait()

        def transpose(s):
            rows_s = rows_v.at[s]
            tile_s = tile_v.at[s]

            def body(k, carry):
                grp = k % (_LANE // 16)
                f = (k // (_LANE // 16)) % d
                u = k // ((_LANE // 16) * d)
                bl0 = grp * 16
                idx_r = (u * _LANE + bl0) + io
                idx_c = f + io * 0
                v = plsc.load_gather(rows_s, [idx_r, idx_c])
                woff = ((f // _SUB) * blk + u * tile_words
                        + (f % _SUB) * _LANE + bl0)
                tile_s[pl.ds(pl.multiple_of(woff, 16), 16)] = v
                return carry

            lax.fori_loop(0, (_LANE // 16) * d * units, body, 0)

        def issue_write(c, s):
            j0 = base + c * _CHUNK
            t = j0 // n_b
            bb0 = (j0 % n_b) // _LANE
            blk = units * tile_words
            for fb in range(n_fb):
                dst = pl.multiple_of(((t * n_fb + fb) * (n_b // _LANE) + bb0)
                                     * tile_words, 8)
                pltpu.async_copy(
                    tile_v.at[s].at[pl.ds(fb * blk, blk)],
                    out_hbm.at[pl.ds(dst, blk)], wsem[s])

        def wait_write(s):
            pltpu.make_async_copy(
                tile_v.at[s], out_hbm.at[pl.ds(0, _CHUNK * d)], wsem[s]).wait()

        issue_gather(0, 0)

        def body(i, carry):
            for s in (0, 1):
                c = 2 * i + s
                wait_gather(s)
                @pl.when(c + 1 < n_chunks)
                def _():
                    issue_gather(c + 1, 1 - s)
                @pl.when(c >= 2)
                def _():
                    wait_write(s)
                transpose(s)
                issue_write(c, s)
            return carry

        lax.fori_loop(0, n_chunks // 2, body, 0)
        wait_write(0)
        wait_write(1)

    return gather_kernel


def kernel(X, weight):
    n_b, n_t = X.shape
    d = weight.shape[1]
    idx_t = jnp.swapaxes(X, 0, 1).reshape(-1)
    out1d = _make_gather(n_b, n_t, d)(idx_t, weight)
    arr5 = out1d.reshape(n_t, d // _SUB, n_b // _LANE, _SUB, _LANE)
    return arr5.transpose(2, 4, 0, 1, 3).reshape(n_b, n_t, d)


# speedup vs baseline: 1.0009x; 1.0009x over previous
import functools

import jax
import jax.numpy as jnp
from jax import lax
from jax.experimental import pallas as pl
from jax.experimental.pallas import tpu as pltpu
from jax.experimental.pallas import tpu_sc as plsc

_NUM_CORES = 2
_NUM_SUBCORES = 16
_NUM_WORKERS = _NUM_CORES * _NUM_SUBCORES
_CHUNK = 512
_LANE = 128
_SUB = 8


@functools.lru_cache(maxsize=None)
def _make_gather(n_b, n_t, d):
    n = n_b * n_t
    per_w = n // _NUM_WORKERS
    n_chunks = per_w // _CHUNK
    assert n_chunks % 2 == 0
    assert n_b % _LANE == 0 and d % _SUB == 0 and _CHUNK % _LANE == 0
    n_fb = d // _SUB
    tile_words = _SUB * _LANE
    units = _CHUNK // _LANE
    mesh = plsc.VectorSubcoreMesh(
        core_axis_name="c",
        subcore_axis_name="s",
        num_cores=_NUM_CORES,
        num_subcores=_NUM_SUBCORES,
    )

    @functools.partial(
        pl.kernel,
        mesh=mesh,
        compiler_params=pltpu.CompilerParams(
            use_tc_tiling_on_sc=False, needs_layout_passes=False),
        out_type=jax.ShapeDtypeStruct((n * d,), jnp.float32),
        scratch_types=[
            pltpu.VMEM((per_w,), jnp.int32),
            pltpu.VMEM((2, _CHUNK, d), jnp.float32),
            pltpu.VMEM((2, _CHUNK * d), jnp.float32),
            pltpu.SemaphoreType.DMA,
            pltpu.SemaphoreType.DMA,
            pltpu.SemaphoreType.DMA,
            pltpu.SemaphoreType.DMA,
        ],
    )
    def gather_kernel(idx_hbm, table_hbm, out_hbm, idx_v, rows_v, tile_v,
                      g0, g1, w0, w1):
        gsem = (g0, g1)
        wsem = (w0, w1)
        wid = lax.axis_index("s") * _NUM_CORES + lax.axis_index("c")
        base = wid * per_w

        pltpu.sync_copy(idx_hbm.at[pl.ds(pl.multiple_of(base, 8), per_w)], idx_v)

        io = lax.iota(jnp.int32, 16)
        blk = units * tile_words

        def issue_gather(c, s):
            pltpu.async_copy(
                table_hbm.at[idx_v.at[pl.ds(c * _CHUNK, _CHUNK)]],
                rows_v.at[s], gsem[s])

        def wait_gather(s):
            pltpu.make_async_copy(
                table_hbm.at[idx_v.at[pl.ds(0, _CHUNK)]], rows_v.at[s], gsem[s]
            ).wait()

        def transpose(s):
            rows_s = rows_v.at[s]
            tile_s = tile_v.at[s]

            def body(f, carry):
                idx_c = f + io * 0
                wbase = (f // _SUB) * blk + (f % _SUB) * _LANE
                for u in range(units):
                    for grp in range(_LANE // 16):
                        idx_r = (u * _LANE + grp * 16) + io
                        v = plsc.load_gather(rows_s, [idx_r, idx_c])
                        woff = wbase + u * tile_words + grp * 16
                        tile_s[pl.ds(pl.multiple_of(woff, 16), 16)] = v
                return carry

            lax.fori_loop(0, d, body, 0)

        def issue_write(c, s):
            j0 = base + c * _CHUNK
            t = j0 // n_b
            bb0 = (j0 % n_b) // _LANE
            blk = units * tile_words
            for fb in range(n_fb):
                dst = pl.multiple_of(((t * n_fb + fb) * (n_b // _LANE) + bb0)
                                     * tile_words, 8)
                pltpu.async_copy(
                    tile_v.at[s].at[pl.ds(fb * blk, blk)],
                    out_hbm.at[pl.ds(dst, blk)], wsem[s])

        def wait_write(s):
            pltpu.make_async_copy(
                tile_v.at[s], out_hbm.at[pl.ds(0, _CHUNK * d)], wsem[s]).wait()

        issue_gather(0, 0)

        def body(i, carry):
            for s in (0, 1):
                c = 2 * i + s
                wait_gather(s)
                @pl.when(c + 1 < n_chunks)
                def _():
                    issue_gather(c + 1, 1 - s)
                @pl.when(c >= 2)
                def _():
                    wait_write(s)
                transpose(s)
                issue_write(c, s)
            return carry

        lax.fori_loop(0, n_chunks // 2, body, 0)
        wait_write(0)
        wait_write(1)

    return gather_kernel


def kernel(X, weight):
    n_b, n_t = X.shape
    d = weight.shape[1]
    idx_t = jnp.swapaxes(X, 0, 1).reshape(-1)
    out1d = _make_gather(n_b, n_t, d)(idx_t, weight)
    arr5 = out1d.reshape(n_t, d // _SUB, n_b // _LANE, _SUB, _LANE)
    return arr5.transpose(2, 4, 0, 1, 3).reshape(n_b, n_t, d)


# speedup vs baseline: 1.3420x; 1.3408x over previous
import functools

import jax
import jax.numpy as jnp
from jax import lax
from jax.experimental import pallas as pl
from jax.experimental.pallas import tpu as pltpu
from jax.experimental.pallas import tpu_sc as plsc

_NUM_CORES = 2
_NUM_SUBCORES = 16
_NUM_WORKERS = _NUM_CORES * _NUM_SUBCORES
_CHUNK = 512
_LANE = 128
_SUB = 8


@functools.lru_cache(maxsize=None)
def _make_gather(n_b, n_t, d):
    n = n_b * n_t
    per_w = n // _NUM_WORKERS
    n_chunks = per_w // _CHUNK
    assert n_chunks % 2 == 0
    assert n_b % _LANE == 0 and d % _SUB == 0 and _CHUNK % _LANE == 0
    n_fb = d // _SUB
    tile_words = _SUB * _LANE
    units = _CHUNK // _LANE
    mesh = plsc.VectorSubcoreMesh(
        core_axis_name="c",
        subcore_axis_name="s",
        num_cores=_NUM_CORES,
        num_subcores=_NUM_SUBCORES,
    )

    @functools.partial(
        pl.kernel,
        mesh=mesh,
        compiler_params=pltpu.CompilerParams(
            use_tc_tiling_on_sc=False, needs_layout_passes=False),
        out_type=jax.ShapeDtypeStruct((n * d,), jnp.float32),
        scratch_types=[
            pltpu.VMEM((per_w,), jnp.int32),
            pltpu.VMEM((2, _CHUNK, d), jnp.float32),
            pltpu.VMEM((2, _CHUNK * d), jnp.float32),
            pltpu.SemaphoreType.DMA,
            pltpu.SemaphoreType.DMA,
            pltpu.SemaphoreType.DMA,
            pltpu.SemaphoreType.DMA,
        ],
    )
    def gather_kernel(idx_hbm, table_hbm, out_hbm, idx_v, rows_v, tile_v,
                      g0, g1, w0, w1):
        gsem = (g0, g1)
        wsem = (w0, w1)
        wid = lax.axis_index("s") * _NUM_CORES + lax.axis_index("c")
        base = wid * per_w

        pltpu.sync_copy(idx_hbm.at[pl.ds(pl.multiple_of(base, 8), per_w)], idx_v)

        io = lax.iota(jnp.int32, 16)
        blk = units * tile_words

        def issue_gather(c, s):
            pltpu.async_copy(
                table_hbm.at[idx_v.at[pl.ds(c * _CHUNK, _CHUNK)]],
                rows_v.at[s], gsem[s])

        def wait_gather(s):
            pltpu.make_async_copy(
                table_hbm.at[idx_v.at[pl.ds(0, _CHUNK)]], rows_v.at[s], gsem[s]
            ).wait()

        def transpose(s):
            rows_s = rows_v.at[s]
            tile_s = tile_v.at[s]

            @plsc.parallel_loop(0, d, unroll=2)
            def body(f):
                idx_c = f + io * 0
                wbase = (f // _SUB) * blk + (f % _SUB) * _LANE
                for u in range(units):
                    for grp in range(_LANE // 16):
                        idx_r = (u * _LANE + grp * 16) + io
                        v = plsc.load_gather(rows_s, [idx_r, idx_c])
                        woff = wbase + u * tile_words + grp * 16
                        tile_s[pl.ds(pl.multiple_of(woff, 16), 16)] = v

        def issue_write(c, s):
            j0 = base + c * _CHUNK
            t = j0 // n_b
            bb0 = (j0 % n_b) // _LANE
            blk = units * tile_words
            for fb in range(n_fb):
                dst = pl.multiple_of(((t * n_fb + fb) * (n_b // _LANE) + bb0)
                                     * tile_words, 8)
                pltpu.async_copy(
                    tile_v.at[s].at[pl.ds(fb * blk, blk)],
                    out_hbm.at[pl.ds(dst, blk)], wsem[s])

        def wait_write(s):
            pltpu.make_async_copy(
                tile_v.at[s], out_hbm.at[pl.ds(0, _CHUNK * d)], wsem[s]).wait()

        issue_gather(0, 0)

        def body(i, carry):
            for s in (0, 1):
                c = 2 * i + s
                wait_gather(s)
                @pl.when(c + 1 < n_chunks)
                def _():
                    issue_gather(c + 1, 1 - s)
                @pl.when(c >= 2)
                def _():
                    wait_write(s)
                transpose(s)
                issue_write(c, s)
            return carry

        lax.fori_loop(0, n_chunks // 2, body, 0)
        wait_write(0)
        wait_write(1)

    return gather_kernel


def kernel(X, weight):
    n_b, n_t = X.shape
    d = weight.shape[1]
    idx_t = jnp.swapaxes(X, 0, 1).reshape(-1)
    out1d = _make_gather(n_b, n_t, d)(idx_t, weight)
    arr5 = out1d.reshape(n_t, d // _SUB, n_b // _LANE, _SUB, _LANE)
    return arr5.transpose(2, 4, 0, 1, 3).reshape(n_b, n_t, d)


# speedup vs baseline: 1.3472x; 1.0039x over previous
import functools

import jax
import jax.numpy as jnp
from jax import lax
from jax.experimental import pallas as pl
from jax.experimental.pallas import tpu as pltpu
from jax.experimental.pallas import tpu_sc as plsc

_NUM_CORES = 2
_NUM_SUBCORES = 16
_NUM_WORKERS = _NUM_CORES * _NUM_SUBCORES
_CHUNK = 512
_LANE = 128
_SUB = 8


@functools.lru_cache(maxsize=None)
def _make_gather(n_b, n_t, d):
    n = n_b * n_t
    per_w = n // _NUM_WORKERS
    n_chunks = per_w // _CHUNK
    assert n_chunks % 2 == 0
    assert n_b % _LANE == 0 and d % _SUB == 0 and _CHUNK % _LANE == 0
    n_fb = d // _SUB
    tile_words = _SUB * _LANE
    units = _CHUNK // _LANE
    mesh = plsc.VectorSubcoreMesh(
        core_axis_name="c",
        subcore_axis_name="s",
        num_cores=_NUM_CORES,
        num_subcores=_NUM_SUBCORES,
    )

    @functools.partial(
        pl.kernel,
        mesh=mesh,
        compiler_params=pltpu.CompilerParams(
            use_tc_tiling_on_sc=False, needs_layout_passes=False),
        out_type=jax.ShapeDtypeStruct((n * d,), jnp.float32),
        scratch_types=[
            pltpu.VMEM((per_w,), jnp.int32),
            pltpu.VMEM((2, _CHUNK, d), jnp.float32),
            pltpu.VMEM((2, _CHUNK * d), jnp.float32),
            pltpu.SemaphoreType.DMA,
            pltpu.SemaphoreType.DMA,
            pltpu.SemaphoreType.DMA,
            pltpu.SemaphoreType.DMA,
        ],
    )
    def gather_kernel(idx_hbm, table_hbm, out_hbm, idx_v, rows_v, tile_v,
                      g0, g1, w0, w1):
        gsem = (g0, g1)
        wsem = (w0, w1)
        wid = lax.axis_index("s") * _NUM_CORES + lax.axis_index("c")
        base = wid * per_w

        pltpu.sync_copy(idx_hbm.at[pl.ds(pl.multiple_of(base, 8), per_w)], idx_v)

        io = lax.iota(jnp.int32, 16)
        blk = units * tile_words

        def issue_gather(c, s):
            pltpu.async_copy(
                table_hbm.at[idx_v.at[pl.ds(c * _CHUNK, _CHUNK)]],
                rows_v.at[s], gsem[s])

        def wait_gather(s):
            pltpu.make_async_copy(
                table_hbm.at[idx_v.at[pl.ds(0, _CHUNK)]], rows_v.at[s], gsem[s]
            ).wait()

        def transpose(s):
            rows_s = rows_v.at[s]
            tile_s = tile_v.at[s]

            @plsc.parallel_loop(0, d, unroll=4)
            def body(f):
                idx_c = f + io * 0
                wbase = (f // _SUB) * blk + (f % _SUB) * _LANE
                for u in range(units):
                    for grp in range(_LANE // 16):
                        idx_r = (u * _LANE + grp * 16) + io
                        v = plsc.load_gather(rows_s, [idx_r, idx_c])
                        woff = wbase + u * tile_words + grp * 16
                        tile_s[pl.ds(pl.multiple_of(woff, 16), 16)] = v

        def issue_write(c, s):
            j0 = base + c * _CHUNK
            t = j0 // n_b
            bb0 = (j0 % n_b) // _LANE
            blk = units * tile_words
            for fb in range(n_fb):
                dst = pl.multiple_of(((t * n_fb + fb) * (n_b // _LANE) + bb0)
                                     * tile_words, 8)
                pltpu.async_copy(
                    tile_v.at[s].at[pl.ds(fb * blk, blk)],
                    out_hbm.at[pl.ds(dst, blk)], wsem[s])

        def wait_write(s):
            pltpu.make_async_copy(
                tile_v.at[s], out_hbm.at[pl.ds(0, _CHUNK * d)], wsem[s]).wait()

        issue_gather(0, 0)

        def body(i, carry):
            for s in (0, 1):
                c = 2 * i + s
                wait_gather(s)
                @pl.when(c + 1 < n_chunks)
                def _():
                    issue_gather(c + 1, 1 - s)
                @pl.when(c >= 2)
                def _():
                    wait_write(s)
                transpose(s)
                issue_write(c, s)
            return carry

        lax.fori_loop(0, n_chunks // 2, body, 0)
        wait_write(0)
        wait_write(1)

    return gather_kernel


def kernel(X, weight):
    n_b, n_t = X.shape
    d = weight.shape[1]
    idx_t = jnp.swapaxes(X, 0, 1).reshape(-1)
    out1d = _make_gather(n_b, n_t, d)(idx_t, weight)
    arr5 = out1d.reshape(n_t, d // _SUB, n_b // _LANE, _SUB, _LANE)
    return arr5.transpose(2, 4, 0, 1, 3).reshape(n_b, n_t, d)
